# baseline (speedup 1.0000x reference)
import functools

import jax
import jax.numpy as jnp
from jax import lax
from jax.experimental import pallas as pl
from jax.experimental.pallas import tpu as pltpu
from jax.experimental.pallas import tpu_sc as plsc

B = 16384
CTX = 20
NT = 5
H = 64
NC = 2
NS = 16
L = 16
NW = NC * NS
BPW = B // NW
HALF = BPW // 2
S1_ROWS = 32
S1_STEPS = HALF // S1_ROWS
S2_ROWS = 128
S2_STEPS = HALF // S2_ROWS
GROWS = 640
HQ = H // L


def _dot_rows(conbuf, gbuf, buf, iota, row_base, g):
  def body(rr, yv):
    row = row_base + rr
    cv = [conbuf[row, pl.ds(h * L, L)] for h in range(HQ)]
    tbase = (g * L + rr) * NT
    out = []
    for j in range(NT):
      d = cv[0] * gbuf[buf, tbase + j, pl.ds(0, L)]
      for h in range(1, HQ):
        d = d + cv[h] * gbuf[buf, tbase + j, pl.ds(h * L, L)]
      s = jnp.sum(d)
      out.append(jnp.where(iota == rr, s, yv[j]))
    return tuple(out)
  zeros = jnp.zeros((L,), jnp.float32)
  return lax.fori_loop(0, L, body, (zeros,) * NT, unroll=False)


def kernel(contexts, t, in_emb, out_emb):
  ctx_r = contexts.reshape(NW, (BPW * CTX) // 128, 128)
  t_r = t.reshape(NW, (BPW * NT) // 128, 128)

  mesh = plsc.VectorSubcoreMesh(core_axis_name="c", subcore_axis_name="s")

  @functools.partial(
      pl.kernel,
      out_type=jax.ShapeDtypeStruct((NW, BPW * NT), jnp.float32),
      mesh=mesh,
      compiler_params=pltpu.CompilerParams(
          needs_layout_passes=False, use_tc_tiling_on_sc=False),
      scratch_types=[
          pltpu.VMEM((BPW * CTX // 128, 128), jnp.int32),
          pltpu.VMEM((BPW * NT // 128, 128), jnp.int32),
          pltpu.VMEM((2, GROWS, H), jnp.float32),
          pltpu.VMEM((HALF, H), jnp.float32),
          pltpu.VMEM((BPW * NT,), jnp.float32),
          pltpu.SemaphoreType.DMA,
          pltpu.SemaphoreType.DMA,
      ],
  )
  def run(ctx_hbm, t_hbm, in_hbm, oute_hbm, y_hbm,
          ctx_idx, t_idx, gbuf, conbuf, ybuf, sem0, sem1):
    wid = lax.axis_index("s") * NC + lax.axis_index("c")
    sems = (sem0, sem1)
    iota = lax.iota(jnp.int32, L)

    pltpu.sync_copy(ctx_hbm.at[wid], ctx_idx)
    pltpu.sync_copy(t_hbm.at[wid], t_idx)

    def fire(table, idx, step, buf):
      for j in range(5):
        pltpu.async_copy(
            table.at[idx.at[step * 5 + j]],
            gbuf.at[buf, pl.ds(128 * j, 128)],
            sems[buf])

    def wait(buf):
      pltpu.make_async_copy(
          in_hbm.at[pl.ds(0, GROWS)], gbuf.at[buf], sems[buf]).wait()

    def consum_step(step_in_half, buf):
      row0 = step_in_half * S1_ROWS
      def body(b, _):
        base = b * CTX
        acc = [gbuf[buf, base, pl.ds(h * L, L)] for h in range(HQ)]
        for c in range(1, CTX):
          for h in range(HQ):
            acc[h] = acc[h] + gbuf[buf, base + c, pl.ds(h * L, L)]
        for h in range(HQ):
          conbuf[row0 + b, pl.ds(h * L, L)] = acc[h]
        return 0
      lax.fori_loop(0, S1_ROWS, body, 0, unroll=False)

    def dot_step(half, step_in_half, buf):
      def body(g, _):
        row_base = step_in_half * S2_ROWS + g * L
        yv = _dot_rows(conbuf, gbuf, buf, iota, row_base, g)
        out_base = (half * HALF + row_base) * NT
        for j in range(NT):
          plsc.store_scatter(ybuf, [iota * NT + (out_base + j)], yv[j])
        return 0
      lax.fori_loop(0, S2_ROWS // L, body, 0, unroll=False)

    for half in range(2):
      base1 = half * S1_STEPS
      fire(in_hbm, ctx_idx, base1, 0)
      def pair1(p, _, base1=base1):
        for b in range(2):
          s = p * 2 + b
          wait(b)
          @pl.when(s + 1 < S1_STEPS)
          def _():
            fire(in_hbm, ctx_idx, base1 + s + 1, (b + 1) % 2)
          consum_step(s, b)
        return 0
      lax.fori_loop(0, S1_STEPS // 2, pair1, 0, unroll=False)

      base2 = half * S2_STEPS
      fire(oute_hbm, t_idx, base2, 0)
      for s2 in range(S2_STEPS):
        wait(s2)
        if s2 + 1 < S2_STEPS:
          fire(oute_hbm, t_idx, base2 + s2 + 1, s2 + 1)
        dot_step(half, s2, s2)

    pltpu.sync_copy(ybuf, y_hbm.at[wid])

  y = run(ctx_r, t_r, in_emb, out_emb)
  return y.reshape(B, NT)
